# Initial kernel scaffold; baseline (speedup 1.0000x reference)
import jax
import jax.numpy as jnp
from jax.experimental import pallas as pl


def kernel(x, edge_index, W0, a_src0, a_dst0, b0, W1, a_src1, a_dst1, b1):
    raise NotImplementedError("write your pallas kernel here")


# speedup vs baseline: 27.9319x; 27.9319x over previous
import functools

import jax
import jax.numpy as jnp
from jax import lax
from jax.experimental import pallas as pl
from jax.experimental.pallas import tpu as pltpu
from jax.experimental.pallas import tpu_sc as plsc

N = 10000
NP = 10240
E_REAL = 330000
B = 128
KBT = 176
NBT = 16 * KBT
EPAD = NBT * B
RPT = NP // 16

W1T = 80
W2T = 48
RN = 512
GRID = NP // RN


def _k1_body(x_ref, w0_ref, a0_ref, tab_ref, adst_ref, m_ref):
    t = pl.program_id(0)
    i = pl.program_id(1)
    h = jnp.dot(x_ref[...], w0_ref[...], preferred_element_type=jnp.float32)
    aa = jnp.dot(h, a0_ref[...], preferred_element_type=jnp.float32)
    hsel = jnp.where(t == 0, h[:, 0:64], h[:, 64:128])
    asel = jnp.where(t == 0, aa[:, 0:4], aa[:, 4:8])
    tab_ref[0] = jnp.concatenate(
        [hsel, asel, jnp.zeros((RN, 12), jnp.float32)], axis=1)
    adst_ref[...] = aa[:, 8:16]
    mloc = jnp.max(aa, axis=0, keepdims=True)

    @pl.when((t == 0) & (i == 0))
    def _():
        m_ref[...] = mloc

    @pl.when((t > 0) | (i > 0))
    def _():
        m_ref[...] = jnp.maximum(m_ref[...], mloc)


def _k1(xp, W0, A0):
    return pl.pallas_call(
        _k1_body,
        grid=(2, GRID),
        in_specs=[
            pl.BlockSpec((RN, 128), lambda t, i: (i, 0)),
            pl.BlockSpec((128, 128), lambda t, i: (0, 0)),
            pl.BlockSpec((128, 16), lambda t, i: (0, 0)),
        ],
        out_specs=[
            pl.BlockSpec((1, RN, W1T), lambda t, i: (t, i, 0)),
            pl.BlockSpec((RN, 8), lambda t, i: (i, 0)),
            pl.BlockSpec((1, 16), lambda t, i: (0, 0)),
        ],
        out_shape=[
            jax.ShapeDtypeStruct((2, NP, W1T), jnp.float32),
            jax.ShapeDtypeStruct((NP, 8), jnp.float32),
            jax.ShapeDtypeStruct((1, 16), jnp.float32),
        ],
    )(xp, W0, A0)


def _k3_body(pa_ref, pb_ref, b0_ref, w1_ref, a1_ref, ex_ref,
             tab_ref, adst_ref, m_ref):
    t = pl.program_id(0)
    i = pl.program_id(1)
    den8 = jnp.concatenate([pa_ref[:, 64:68], pb_ref[:, 64:68]], axis=1)
    den = jnp.dot(den8, ex_ref[...], preferred_element_type=jnp.float32)
    h128 = jnp.concatenate([pa_ref[:, 0:64], pb_ref[:, 0:64]], axis=1)
    o = h128 / (den + 1e-16) + b0_ref[...]
    o = jnp.where(o > 0, o, jnp.exp(o) - 1.0)
    h1 = jnp.dot(o, w1_ref[...], preferred_element_type=jnp.float32)
    aa = jnp.dot(h1, a1_ref[...], preferred_element_type=jnp.float32)
    hsel = jnp.where(t == 0, h1[:, 0:32], h1[:, 32:64])
    tab_ref[0] = jnp.concatenate(
        [hsel, aa[:, 0:1], jnp.zeros((RN, 15), jnp.float32)], axis=1)
    adst_ref[...] = jnp.concatenate([aa[:, 1:2]] * 8, axis=1)
    mloc = jnp.max(aa, axis=0, keepdims=True)

    @pl.when((t == 0) & (i == 0))
    def _():
        m_ref[...] = mloc

    @pl.when((t > 0) | (i > 0))
    def _():
        m_ref[...] = jnp.maximum(m_ref[...], mloc)


def _k3(pa, pb, b0r, W1, A1, EX8):
    return pl.pallas_call(
        _k3_body,
        grid=(2, GRID),
        in_specs=[
            pl.BlockSpec((RN, W1T), lambda t, i: (i, 0)),
            pl.BlockSpec((RN, W1T), lambda t, i: (i, 0)),
            pl.BlockSpec((1, 128), lambda t, i: (0, 0)),
            pl.BlockSpec((128, 64), lambda t, i: (0, 0)),
            pl.BlockSpec((64, 16), lambda t, i: (0, 0)),
            pl.BlockSpec((8, 128), lambda t, i: (0, 0)),
        ],
        out_specs=[
            pl.BlockSpec((1, RN, W2T), lambda t, i: (t, i, 0)),
            pl.BlockSpec((RN, 8), lambda t, i: (i, 0)),
            pl.BlockSpec((1, 16), lambda t, i: (0, 0)),
        ],
        out_shape=[
            jax.ShapeDtypeStruct((2, NP, W2T), jnp.float32),
            jax.ShapeDtypeStruct((NP, 8), jnp.float32),
            jax.ShapeDtypeStruct((1, 16), jnp.float32),
        ],
    )(pa, pb, b0r, W1, A1, EX8)


def _k5_body(pa_ref, pb_ref, b1_ref, ex_ref, out_ref):
    s = jnp.concatenate([pa_ref[:, 0:32], pb_ref[:, 0:32]], axis=1)
    den = jnp.dot(pa_ref[:, 32:40], ex_ref[...],
                  preferred_element_type=jnp.float32)
    out_ref[...] = s / (den + 1e-16) + b1_ref[...]


def _k5(pa, pb, b1r, EX2):
    return pl.pallas_call(
        _k5_body,
        grid=(GRID,),
        in_specs=[
            pl.BlockSpec((RN, W2T), lambda i: (i, 0)),
            pl.BlockSpec((RN, W2T), lambda i: (i, 0)),
            pl.BlockSpec((1, 64), lambda i: (0, 0)),
            pl.BlockSpec((8, 64), lambda i: (0, 0)),
        ],
        out_specs=pl.BlockSpec((RN, 64), lambda i: (i, 0)),
        out_shape=jax.ShapeDtypeStruct((NP, 64), jnp.float32),
    )(pa, pb, b1r, EX2)


def _make_edge_pass(wtab, nh, acol):
    mesh = plsc.VectorSubcoreMesh(core_axis_name="c", subcore_axis_name="s")
    nchunk = (acol // nh) // 16

    @functools.partial(
        pl.kernel,
        out_type=jax.ShapeDtypeStruct((2, NP, wtab), jnp.float32),
        mesh=mesh,
        compiler_params=pltpu.CompilerParams(
            use_tc_tiling_on_sc=False, needs_layout_passes=False),
        scratch_types=[
            pltpu.VMEM_SHARED((NP, wtab), jnp.float32),
            pltpu.VMEM((B, wtab), jnp.float32),
            pltpu.VMEM((B, wtab), jnp.float32),
            pltpu.VMEM((B, 8), jnp.float32),
            pltpu.VMEM((B, 8), jnp.float32),
            pltpu.VMEM((KBT, B), jnp.int32),
            pltpu.VMEM((KBT, B), jnp.int32),
            pltpu.VMEM((nh, 16), jnp.float32),
            pltpu.SemaphoreType.DMA,
            pltpu.SemaphoreType.DMA,
            pltpu.SemaphoreType.DMA,
            pltpu.SemaphoreType.DMA,
        ],
    )
    def edge_pass(tab_hbm, adst_hbm, srcb_hbm, dstb_hbm, mp_hbm, z_hbm,
                  out_hbm, acc, g0, g1, d0, d1, idxs, idxd, mp,
                  sg0, sg1, sd0, sd1):
        c = lax.axis_index("c")
        s = lax.axis_index("s")
        g = (g0, g1)
        d = (d0, d1)
        sg = (sg0, sg1)
        sd = (sd0, sd1)

        for t in range(RPT // B):
            pltpu.sync_copy(z_hbm, acc.at[pl.ds(s * RPT + t * B, B)])

        pltpu.sync_copy(srcb_hbm.at[c, pl.ds(s * KBT, KBT)], idxs)
        pltpu.sync_copy(dstb_hbm.at[pl.ds(s * KBT, KBT)], idxd)
        pltpu.sync_copy(mp_hbm.at[c], mp)
        plsc.subcore_barrier()

        pltpu.async_copy(tab_hbm.at[idxs.at[0]], g0, sg0)
        pltpu.async_copy(adst_hbm.at[idxd.at[0]], d0, sd0)

        iot = lax.iota(jnp.int32, 16)

        def _compute(p, jb):
            for eg in range(8):
                rows = iot + eg * 16
                for h in range(nh):
                    ca = jnp.full((16,), acol + h, jnp.int32)
                    cd = jnp.full((16,), h, jnp.int32) + nh * c
                    asrc = plsc.load_gather(g[p], [rows, ca])
                    adst = plsc.load_gather(d[p], [rows, cd])
                    e = asrc + adst
                    e = jnp.where(e >= 0.0, e, 0.2 * e)
                    w = jnp.exp(e - mp[h, :])
                    plsc.store_scatter(g[p], [rows, ca], w)

            def _mb(b, _):
                wrow = g[p][b, pl.ds(acol, 16)]
                for h in range(nh):
                    wv = wrow[h]
                    for cc in range(nchunk):
                        slc = pl.ds((h * nchunk + cc) * 16, 16)
                        g[p][b, slc] = g[p][b, slc] * wv
                return 0

            lax.fori_loop(0, B, _mb, 0)
            pltpu.sync_copy(g[p], acc.at[idxd.at[jb]], add=True)

        def _iter(jj, _):
            for p in range(2):
                jb = jj * 2 + p
                q = 1 - p
                nxt = jb + 1

                @pl.when(nxt < KBT)
                def _():
                    pltpu.async_copy(tab_hbm.at[idxs.at[nxt]], g[q], sg[q])
                    pltpu.async_copy(adst_hbm.at[idxd.at[nxt]], d[q], sd[q])

                pltpu.make_async_copy(tab_hbm.at[idxs.at[0]], g[p], sg[p]).wait()
                pltpu.make_async_copy(adst_hbm.at[idxd.at[0]], d[p], sd[p]).wait()
                _compute(p, jb)
            return 0

        lax.fori_loop(0, KBT // 2, _iter, 0)

        plsc.subcore_barrier()
        for t in range(RPT // B):
            r0 = s * RPT + t * B
            pltpu.sync_copy(acc.at[pl.ds(r0, B)], out_hbm.at[c, pl.ds(r0, B)])

    return edge_pass


_edge_pass_1 = _make_edge_pass(W1T, 4, 64)
_edge_pass_2 = _make_edge_pass(W2T, 1, 32)


def _leaky(v):
    return jnp.where(v >= 0.0, v, 0.2 * v)


def kernel(x, edge_index, W0, a_src0, a_dst0, b0, W1, a_src1, a_dst1, b1):
    f32 = jnp.float32
    xp = jnp.pad(x.astype(f32), ((0, NP - N), (0, 0)))

    eye8 = jnp.eye(8, dtype=f32)
    A0s = (eye8[:, None, :] * a_src0[0][:, :, None]).reshape(128, 8)
    A0d = (eye8[:, None, :] * a_dst0[0][:, :, None]).reshape(128, 8)
    A0 = jnp.concatenate([A0s, A0d], axis=1)
    A1 = jnp.concatenate(
        [a_src1.reshape(64, 1), a_dst1.reshape(64, 1),
         jnp.zeros((64, 14), f32)], axis=1)

    ei = edge_index.astype(jnp.int32)
    loops = jnp.arange(N, dtype=jnp.int32)
    padv = jnp.full((EPAD - E_REAL,), N, jnp.int32)
    src = jnp.concatenate([ei[0], loops, padv])
    dst = jnp.concatenate([ei[1], loops, padv])
    srcb = jnp.stack([src, src + NP]).reshape(2, NBT, B)
    dstb = dst.reshape(NBT, B)

    i8 = jnp.arange(8)
    EX8 = (i8[:, None] == (jnp.arange(128)[None, :] // 16)).astype(f32)
    EX2 = (i8[:, None] == 0).astype(f32) * jnp.ones((1, 64), f32)

    z1 = jnp.zeros((B, W1T), f32)
    z2 = jnp.zeros((B, W2T), f32)

    tab1, adst0, m0 = _k1(xp, W0, A0)
    mp0 = _leaky(m0[0, 0:8] + m0[0, 8:16])
    mp0r = jnp.broadcast_to(mp0.reshape(2, 4, 1), (2, 4, 16))

    parts1 = _edge_pass_1(tab1.reshape(2 * NP, W1T), adst0, srcb, dstb,
                          mp0r, z1)

    tab2, adst1, m1 = _k3(parts1[0], parts1[1], b0.reshape(1, 128),
                          W1, A1, EX8)
    mp1 = _leaky(m1[0, 0:1] + m1[0, 1:2])
    mp1r = jnp.broadcast_to(mp1.reshape(1, 1, 1), (2, 1, 16))

    parts2 = _edge_pass_2(tab2.reshape(2 * NP, W2T), adst1, srcb, dstb,
                          mp1r, z2)

    out = _k5(parts2[0], parts2[1], b1.reshape(1, 64), EX2)
    return out[:N]


# speedup vs baseline: 35.2369x; 1.2615x over previous
import functools

import jax
import jax.numpy as jnp
from jax import lax
from jax.experimental import pallas as pl
from jax.experimental.pallas import tpu as pltpu
from jax.experimental.pallas import tpu_sc as plsc

N = 10000
NP = 10240
E_REAL = 330000
B = 128
KBT = 168
NBT = 16 * KBT
EPAD = NBT * B
RPT = NP // 16

W1T = 72
W2T = 40
RN = 512
GRID = NP // RN


def _k1_body(x_ref, w0_ref, a0_ref, tab_ref, adst_ref, m_ref):
    t = pl.program_id(0)
    i = pl.program_id(1)
    h = jnp.dot(x_ref[...], w0_ref[...], preferred_element_type=jnp.float32)
    aa = jnp.dot(h, a0_ref[...], preferred_element_type=jnp.float32)
    hsel = jnp.where(t == 0, h[:, 0:64], h[:, 64:128])
    asel = jnp.where(t == 0, aa[:, 0:4], aa[:, 4:8])
    tab_ref[0] = jnp.concatenate(
        [hsel, asel, jnp.zeros((RN, 4), jnp.float32)], axis=1)
    adst_ref[...] = aa[:, 8:16]
    mloc = jnp.max(aa, axis=0, keepdims=True)

    @pl.when((t == 0) & (i == 0))
    def _():
        m_ref[...] = mloc

    @pl.when((t > 0) | (i > 0))
    def _():
        m_ref[...] = jnp.maximum(m_ref[...], mloc)


def _k1(xp, W0, A0):
    return pl.pallas_call(
        _k1_body,
        grid=(2, GRID),
        in_specs=[
            pl.BlockSpec((RN, 128), lambda t, i: (i, 0)),
            pl.BlockSpec((128, 128), lambda t, i: (0, 0)),
            pl.BlockSpec((128, 16), lambda t, i: (0, 0)),
        ],
        out_specs=[
            pl.BlockSpec((1, RN, W1T), lambda t, i: (t, i, 0)),
            pl.BlockSpec((RN, 8), lambda t, i: (i, 0)),
            pl.BlockSpec((1, 16), lambda t, i: (0, 0)),
        ],
        out_shape=[
            jax.ShapeDtypeStruct((2, NP, W1T), jnp.float32),
            jax.ShapeDtypeStruct((NP, 8), jnp.float32),
            jax.ShapeDtypeStruct((1, 16), jnp.float32),
        ],
    )(xp, W0, A0)


def _k3_body(pa_ref, pb_ref, b0_ref, w1_ref, a1_ref, ex_ref,
             tab_ref, adst_ref, m_ref):
    t = pl.program_id(0)
    i = pl.program_id(1)
    den8 = jnp.concatenate([pa_ref[:, 64:68], pb_ref[:, 64:68]], axis=1)
    den = jnp.dot(den8, ex_ref[...], preferred_element_type=jnp.float32)
    h128 = jnp.concatenate([pa_ref[:, 0:64], pb_ref[:, 0:64]], axis=1)
    o = h128 / (den + 1e-16) + b0_ref[...]
    o = jnp.where(o > 0, o, jnp.exp(o) - 1.0)
    h1 = jnp.dot(o, w1_ref[...], preferred_element_type=jnp.float32)
    aa = jnp.dot(h1, a1_ref[...], preferred_element_type=jnp.float32)
    hsel = jnp.where(t == 0, h1[:, 0:32], h1[:, 32:64])
    tab_ref[0] = jnp.concatenate(
        [hsel, aa[:, 0:1], jnp.zeros((RN, 7), jnp.float32)], axis=1)
    adst_ref[...] = jnp.concatenate([aa[:, 1:2]] * 8, axis=1)
    mloc = jnp.max(aa, axis=0, keepdims=True)

    @pl.when((t == 0) & (i == 0))
    def _():
        m_ref[...] = mloc

    @pl.when((t > 0) | (i > 0))
    def _():
        m_ref[...] = jnp.maximum(m_ref[...], mloc)


def _k3(pa, pb, b0r, W1, A1, EX8):
    return pl.pallas_call(
        _k3_body,
        grid=(2, GRID),
        in_specs=[
            pl.BlockSpec((RN, W1T), lambda t, i: (i, 0)),
            pl.BlockSpec((RN, W1T), lambda t, i: (i, 0)),
            pl.BlockSpec((1, 128), lambda t, i: (0, 0)),
            pl.BlockSpec((128, 64), lambda t, i: (0, 0)),
            pl.BlockSpec((64, 16), lambda t, i: (0, 0)),
            pl.BlockSpec((8, 128), lambda t, i: (0, 0)),
        ],
        out_specs=[
            pl.BlockSpec((1, RN, W2T), lambda t, i: (t, i, 0)),
            pl.BlockSpec((RN, 8), lambda t, i: (i, 0)),
            pl.BlockSpec((1, 16), lambda t, i: (0, 0)),
        ],
        out_shape=[
            jax.ShapeDtypeStruct((2, NP, W2T), jnp.float32),
            jax.ShapeDtypeStruct((NP, 8), jnp.float32),
            jax.ShapeDtypeStruct((1, 16), jnp.float32),
        ],
    )(pa, pb, b0r, W1, A1, EX8)


def _k5_body(pa_ref, pb_ref, b1_ref, ex_ref, out_ref):
    s = jnp.concatenate([pa_ref[:, 0:32], pb_ref[:, 0:32]], axis=1)
    den = jnp.dot(pa_ref[:, 32:40], ex_ref[...],
                  preferred_element_type=jnp.float32)
    out_ref[...] = s / (den + 1e-16) + b1_ref[...]


def _k5(pa, pb, b1r, EX2):
    return pl.pallas_call(
        _k5_body,
        grid=(GRID,),
        in_specs=[
            pl.BlockSpec((RN, W2T), lambda i: (i, 0)),
            pl.BlockSpec((RN, W2T), lambda i: (i, 0)),
            pl.BlockSpec((1, 64), lambda i: (0, 0)),
            pl.BlockSpec((8, 64), lambda i: (0, 0)),
        ],
        out_specs=pl.BlockSpec((RN, 64), lambda i: (i, 0)),
        out_shape=jax.ShapeDtypeStruct((NP, 64), jnp.float32),
    )(pa, pb, b1r, EX2)


def _make_edge_pass(wtab, nh, acol):
    mesh = plsc.VectorSubcoreMesh(core_axis_name="c", subcore_axis_name="s")
    NR = 4
    mc = acol // nh

    @functools.partial(
        pl.kernel,
        out_type=jax.ShapeDtypeStruct((2, NP, wtab), jnp.float32),
        mesh=mesh,
        compiler_params=pltpu.CompilerParams(
            use_tc_tiling_on_sc=False, needs_layout_passes=False),
        scratch_types=[
            pltpu.VMEM_SHARED((NP, wtab), jnp.float32),
            [pltpu.VMEM((B, wtab), jnp.float32)] * NR,
            [pltpu.VMEM((B, 8), jnp.float32)] * NR,
            pltpu.VMEM((KBT, B), jnp.int32),
            pltpu.VMEM((KBT, B), jnp.int32),
            pltpu.VMEM((nh, 16), jnp.float32),
            [pltpu.SemaphoreType.DMA] * NR,
            [pltpu.SemaphoreType.DMA] * NR,
            [pltpu.SemaphoreType.DMA] * NR,
        ],
    )
    def edge_pass(tab_hbm, adst_hbm, srcb_hbm, dstb_hbm, mp_hbm, z_hbm,
                  out_hbm, acc, g, d, idxs, idxd, mp, sg, sd, ss):
        c = lax.axis_index("c")
        s = lax.axis_index("s")

        for t in range(RPT // B):
            pltpu.sync_copy(z_hbm, acc.at[pl.ds(s * RPT + t * B, B)])

        pltpu.sync_copy(srcb_hbm.at[c, pl.ds(s * KBT, KBT)], idxs)
        pltpu.sync_copy(dstb_hbm.at[pl.ds(s * KBT, KBT)], idxd)
        pltpu.sync_copy(mp_hbm.at[c], mp)
        plsc.subcore_barrier()

        def _issue_gather(f, blk):
            pltpu.async_copy(tab_hbm.at[idxs.at[blk]], g[f], sg[f])
            pltpu.async_copy(adst_hbm.at[idxd.at[blk]], d[f], sd[f])

        _issue_gather(0, 0)
        _issue_gather(1, 1)

        iot = lax.iota(jnp.int32, 16)

        def _compute(p, jb):
            def _eg(eg, _):
                rows = iot + eg * 16
                for h in range(nh):
                    ca = jnp.full((16,), acol + h, jnp.int32)
                    cd = jnp.full((16,), h, jnp.int32) + nh * c
                    asrc = plsc.load_gather(g[p], [rows, ca])
                    adst = plsc.load_gather(d[p], [rows, cd])
                    e = asrc + adst
                    e = jnp.where(e >= 0.0, e, 0.2 * e)
                    w = jnp.exp(e - mp[h, :])
                    plsc.store_scatter(g[p], [rows, ca], w)
                    for cc in range(mc):
                        col = jnp.full((16,), h * mc + cc, jnp.int32)
                        v = plsc.load_gather(g[p], [rows, col])
                        plsc.store_scatter(g[p], [rows, col], v * w)
                return 0

            lax.fori_loop(0, 8, _eg, 0)
            pltpu.async_copy(g[p], acc.at[idxd.at[jb]], ss[p], add=True)

        def _iter(jj, _):
            for p in range(NR):
                jb = jj * NR + p
                f = (p + 2) % NR
                nxt = jb + 2

                @pl.when((nxt < KBT) & (jb >= 2))
                def _():
                    pltpu.make_async_copy(
                        g[f], acc.at[idxd.at[0]], ss[f]).wait()

                @pl.when(nxt < KBT)
                def _():
                    _issue_gather(f, nxt)

                pltpu.make_async_copy(tab_hbm.at[idxs.at[0]], g[p], sg[p]).wait()
                pltpu.make_async_copy(adst_hbm.at[idxd.at[0]], d[p], sd[p]).wait()
                _compute(p, jb)
            return 0

        lax.fori_loop(0, KBT // NR, _iter, 0)

        for p in range(NR):
            pltpu.make_async_copy(g[p], acc.at[idxd.at[0]], ss[p]).wait()

        plsc.subcore_barrier()
        for t in range(RPT // B):
            r0 = s * RPT + t * B
            pltpu.sync_copy(acc.at[pl.ds(r0, B)], out_hbm.at[c, pl.ds(r0, B)])

    return edge_pass


_edge_pass_1 = _make_edge_pass(W1T, 4, 64)
_edge_pass_2 = _make_edge_pass(W2T, 1, 32)


def _leaky(v):
    return jnp.where(v >= 0.0, v, 0.2 * v)


def kernel(x, edge_index, W0, a_src0, a_dst0, b0, W1, a_src1, a_dst1, b1):
    f32 = jnp.float32
    xp = jnp.pad(x.astype(f32), ((0, NP - N), (0, 0)))

    eye8 = jnp.eye(8, dtype=f32)
    A0s = (eye8[:, None, :] * a_src0[0][:, :, None]).reshape(128, 8)
    A0d = (eye8[:, None, :] * a_dst0[0][:, :, None]).reshape(128, 8)
    A0 = jnp.concatenate([A0s, A0d], axis=1)
    A1 = jnp.concatenate(
        [a_src1.reshape(64, 1), a_dst1.reshape(64, 1),
         jnp.zeros((64, 14), f32)], axis=1)

    ei = edge_index.astype(jnp.int32)
    loops = jnp.arange(N, dtype=jnp.int32)
    padv = jnp.full((EPAD - E_REAL,), N, jnp.int32)
    src = jnp.concatenate([ei[0], loops, padv])
    dst = jnp.concatenate([ei[1], loops, padv])
    srcb = jnp.stack([src, src + NP]).reshape(2, NBT, B)
    dstb = dst.reshape(NBT, B)

    i8 = jnp.arange(8)
    EX8 = (i8[:, None] == (jnp.arange(128)[None, :] // 16)).astype(f32)
    EX2 = (i8[:, None] == 0).astype(f32) * jnp.ones((1, 64), f32)

    z1 = jnp.zeros((B, W1T), f32)
    z2 = jnp.zeros((B, W2T), f32)

    tab1, adst0, m0 = _k1(xp, W0, A0)
    mp0 = _leaky(m0[0, 0:8] + m0[0, 8:16])
    mp0r = jnp.broadcast_to(mp0.reshape(2, 4, 1), (2, 4, 16))

    parts1 = _edge_pass_1(tab1.reshape(2 * NP, W1T), adst0, srcb, dstb,
                          mp0r, z1)

    tab2, adst1, m1 = _k3(parts1[0], parts1[1], b0.reshape(1, 128),
                          W1, A1, EX8)
    mp1 = _leaky(m1[0, 0:1] + m1[0, 1:2])
    mp1r = jnp.broadcast_to(mp1.reshape(1, 1, 1), (2, 1, 16))

    parts2 = _edge_pass_2(tab2.reshape(2 * NP, W2T), adst1, srcb, dstb,
                          mp1r, z2)

    out = _k5(parts2[0], parts2[1], b1.reshape(1, 64), EX2)
    return out[:N]


# speedup vs baseline: 49.2108x; 1.3966x over previous
import functools

import jax
import jax.numpy as jnp
from jax import lax
from jax.experimental import pallas as pl
from jax.experimental.pallas import tpu as pltpu
from jax.experimental.pallas import tpu_sc as plsc

N = 10000
NP = 10240
E_REAL = 330000
B = 128
KBT = 168
NBT = 16 * KBT
EPAD = NBT * B
RPT = NP // 16

W1T = 80
W2T = 48
AW1 = 72
AW2 = 40
RN = 512
GRID = NP // RN


def _k1_body(x_ref, w0_ref, a0_ref, tab_ref, adst_ref, m_ref):
    t = pl.program_id(0)
    i = pl.program_id(1)
    h = jnp.dot(x_ref[...], w0_ref[...], preferred_element_type=jnp.float32)
    aa = jnp.dot(h, a0_ref[...], preferred_element_type=jnp.float32)
    hsel = jnp.where(t == 0, h[:, 0:64], h[:, 64:128])
    asel = jnp.where(t == 0, aa[:, 0:4], aa[:, 4:8])
    tab_ref[0] = jnp.concatenate(
        [hsel, asel, jnp.zeros((RN, 12), jnp.float32)], axis=1)
    adst_ref[...] = aa[:, 8:16]
    mloc = jnp.max(aa, axis=0, keepdims=True)

    @pl.when((t == 0) & (i == 0))
    def _():
        m_ref[...] = mloc

    @pl.when((t > 0) | (i > 0))
    def _():
        m_ref[...] = jnp.maximum(m_ref[...], mloc)


def _k1(xp, W0, A0):
    return pl.pallas_call(
        _k1_body,
        grid=(2, GRID),
        in_specs=[
            pl.BlockSpec((RN, 128), lambda t, i: (i, 0)),
            pl.BlockSpec((128, 128), lambda t, i: (0, 0)),
            pl.BlockSpec((128, 16), lambda t, i: (0, 0)),
        ],
        out_specs=[
            pl.BlockSpec((1, RN, W1T), lambda t, i: (t, i, 0)),
            pl.BlockSpec((RN, 8), lambda t, i: (i, 0)),
            pl.BlockSpec((1, 16), lambda t, i: (0, 0)),
        ],
        out_shape=[
            jax.ShapeDtypeStruct((2, NP, W1T), jnp.float32),
            jax.ShapeDtypeStruct((NP, 8), jnp.float32),
            jax.ShapeDtypeStruct((1, 16), jnp.float32),
        ],
    )(xp, W0, A0)


def _k3_body(pa_ref, pb_ref, b0_ref, w1_ref, a1_ref, ex_ref,
             tab_ref, adst_ref, m_ref):
    t = pl.program_id(0)
    i = pl.program_id(1)
    den8 = jnp.concatenate([pa_ref[:, 64:68], pb_ref[:, 64:68]], axis=1)
    den = jnp.dot(den8, ex_ref[...], preferred_element_type=jnp.float32)
    h128 = jnp.concatenate([pa_ref[:, 0:64], pb_ref[:, 0:64]], axis=1)
    o = h128 / (den + 1e-16) + b0_ref[...]
    o = jnp.where(o > 0, o, jnp.exp(o) - 1.0)
    h1 = jnp.dot(o, w1_ref[...], preferred_element_type=jnp.float32)
    aa = jnp.dot(h1, a1_ref[...], preferred_element_type=jnp.float32)
    hsel = jnp.where(t == 0, h1[:, 0:32], h1[:, 32:64])
    tab_ref[0] = jnp.concatenate(
        [hsel, aa[:, 0:1], jnp.zeros((RN, 15), jnp.float32)], axis=1)
    adst_ref[...] = jnp.concatenate([aa[:, 1:2]] * 8, axis=1)
    mloc = jnp.max(aa, axis=0, keepdims=True)

    @pl.when((t == 0) & (i == 0))
    def _():
        m_ref[...] = mloc

    @pl.when((t > 0) | (i > 0))
    def _():
        m_ref[...] = jnp.maximum(m_ref[...], mloc)


def _k3(pa, pb, b0r, W1, A1, EX8):
    return pl.pallas_call(
        _k3_body,
        grid=(2, GRID),
        in_specs=[
            pl.BlockSpec((RN, AW1), lambda t, i: (i, 0)),
            pl.BlockSpec((RN, AW1), lambda t, i: (i, 0)),
            pl.BlockSpec((1, 128), lambda t, i: (0, 0)),
            pl.BlockSpec((128, 64), lambda t, i: (0, 0)),
            pl.BlockSpec((64, 16), lambda t, i: (0, 0)),
            pl.BlockSpec((8, 128), lambda t, i: (0, 0)),
        ],
        out_specs=[
            pl.BlockSpec((1, RN, W2T), lambda t, i: (t, i, 0)),
            pl.BlockSpec((RN, 8), lambda t, i: (i, 0)),
            pl.BlockSpec((1, 16), lambda t, i: (0, 0)),
        ],
        out_shape=[
            jax.ShapeDtypeStruct((2, NP, W2T), jnp.float32),
            jax.ShapeDtypeStruct((NP, 8), jnp.float32),
            jax.ShapeDtypeStruct((1, 16), jnp.float32),
        ],
    )(pa, pb, b0r, W1, A1, EX8)


def _k5_body(pa_ref, pb_ref, b1_ref, ex_ref, out_ref):
    s = jnp.concatenate([pa_ref[:, 0:32], pb_ref[:, 0:32]], axis=1)
    den = jnp.dot(pa_ref[:, 32:36], ex_ref[...],
                  preferred_element_type=jnp.float32)
    out_ref[...] = s / (den + 1e-16) + b1_ref[...]


def _k5(pa, pb, b1r, EX2):
    return pl.pallas_call(
        _k5_body,
        grid=(GRID,),
        in_specs=[
            pl.BlockSpec((RN, AW2), lambda i: (i, 0)),
            pl.BlockSpec((RN, AW2), lambda i: (i, 0)),
            pl.BlockSpec((1, 64), lambda i: (0, 0)),
            pl.BlockSpec((4, 64), lambda i: (0, 0)),
        ],
        out_specs=pl.BlockSpec((RN, 64), lambda i: (i, 0)),
        out_shape=jax.ShapeDtypeStruct((NP, 64), jnp.float32),
    )(pa, pb, b1r, EX2)


def _make_edge_pass(wacc, wg, nh, acol):
    mesh = plsc.VectorSubcoreMesh(core_axis_name="c", subcore_axis_name="s")
    NR = 4
    NS = 1
    npk = acol // (2 * nh)

    @functools.partial(
        pl.kernel,
        out_type=jax.ShapeDtypeStruct((2, NP, wacc), jnp.float32),
        mesh=mesh,
        compiler_params=pltpu.CompilerParams(
            use_tc_tiling_on_sc=False, needs_layout_passes=False),
        scratch_types=[
            pltpu.VMEM_SHARED((NP, wacc), jnp.float32),
            [pltpu.VMEM((B, wg), jnp.int32)] * NR,
            [pltpu.VMEM((B, wacc), jnp.float32)] * NS,
            [pltpu.VMEM((B, 8), jnp.float32)] * NR,
            pltpu.VMEM((KBT, B), jnp.int32),
            pltpu.VMEM((KBT, B), jnp.int32),
            pltpu.VMEM((nh, 16), jnp.float32),
            [pltpu.SemaphoreType.DMA] * NR,
            [pltpu.SemaphoreType.DMA] * NR,
            [pltpu.SemaphoreType.DMA] * NR,
            [pltpu.SemaphoreType.DMA] * NS,
        ],
    )
    def edge_pass(tab_hbm, adst_hbm, srcb_hbm, dstb_hbm, mp_hbm, z_hbm,
                  out_hbm, acc, g, msg, d, idxs, idxd, mp, sg, sg2, sd, ss):
        c = lax.axis_index("c")
        s = lax.axis_index("s")

        for t in range(RPT // B):
            pltpu.sync_copy(z_hbm, acc.at[pl.ds(s * RPT + t * B, B)])
        for m in range(NS):
            pltpu.sync_copy(z_hbm, msg[m])

        pltpu.sync_copy(srcb_hbm.at[c, pl.ds(s * KBT, KBT)], idxs)
        pltpu.sync_copy(dstb_hbm.at[pl.ds(s * KBT, KBT)], idxd)
        pltpu.sync_copy(mp_hbm.at[c], mp)
        plsc.subcore_barrier()

        def _issue_gather(f, blk):
            pltpu.async_copy(tab_hbm.at[idxs.at[blk]], g[f], sg[f])
            pltpu.async_copy(adst_hbm.at[idxd.at[blk]], d[f], sd[f])

        def _wait_gather(p):
            pltpu.make_async_copy(tab_hbm.at[idxs.at[0]], g[p], sg[p]).wait()
            pltpu.make_async_copy(adst_hbm.at[idxd.at[0]], d[p], sd[p]).wait()

        _issue_gather(0, 0)
        _issue_gather(1, 1)
        _issue_gather(2, 2)

        iot = lax.iota(jnp.int32, 16)
        fmt = plsc.PackFormat.INTERLEAVED

        def _unpk(v):
            return plsc.unpack(plsc.bitcast(v, jnp.bfloat16), format=fmt)

        def _compute(p, m, jb):
            @pl.when(jb >= NS)
            def _():
                pltpu.make_async_copy(
                    msg[m], acc.at[idxd.at[0]], ss[m]).wait()

            def _eg(eg, _):
                rows = iot + eg * 16
                up = [_unpk(plsc.load_gather(
                    g[p], [rows, jnp.full((16,), acol // 2 + k, jnp.int32)]))
                    for k in range((nh + 1) // 2)]
                for h in range(nh):
                    asrc = up[h // 2][h % 2]
                    cd = jnp.full((16,), h, jnp.int32) + nh * c
                    adst = plsc.load_gather(d[p], [rows, cd])
                    e = asrc + adst
                    e = jnp.where(e >= 0.0, e, 0.2 * e)
                    w = jnp.exp(e - mp[h, :])
                    plsc.store_scatter(
                        msg[m], [rows, jnp.full((16,), acol + h, jnp.int32)], w)
                    for k in range(npk):
                        wd = h * npk + k
                        a, b = _unpk(plsc.load_gather(
                            g[p], [rows, jnp.full((16,), wd, jnp.int32)]))
                        plsc.store_scatter(
                            msg[m], [rows, jnp.full((16,), 2 * wd, jnp.int32)],
                            a * w)
                        plsc.store_scatter(
                            msg[m],
                            [rows, jnp.full((16,), 2 * wd + 1, jnp.int32)],
                            b * w)
                return 0

            lax.fori_loop(0, 8, _eg, 0)
            pltpu.async_copy(msg[m], acc.at[idxd.at[jb]], ss[m], add=True)

        def _iter(jj, _):
            for p in range(NR):
                jb = jj * NR + p
                f = (p + 3) % NR
                nxt = jb + 3

                @pl.when(nxt < KBT)
                def _():
                    _issue_gather(f, nxt)

                _wait_gather(p)
                _compute(p, p % NS, jb)
            return 0

        lax.fori_loop(0, KBT // NR, _iter, 0)

        for m in range(NS):
            pltpu.make_async_copy(msg[m], acc.at[idxd.at[0]], ss[m]).wait()

        plsc.subcore_barrier()
        for t in range(RPT // B):
            r0 = s * RPT + t * B
            pltpu.sync_copy(acc.at[pl.ds(r0, B)], out_hbm.at[c, pl.ds(r0, B)])

    return edge_pass


_edge_pass_1 = _make_edge_pass(AW1, W1T // 2, 4, 64)
_edge_pass_2 = _make_edge_pass(AW2, W2T // 2, 1, 32)


def _leaky(v):
    return jnp.where(v >= 0.0, v, 0.2 * v)


def kernel(x, edge_index, W0, a_src0, a_dst0, b0, W1, a_src1, a_dst1, b1):
    f32 = jnp.float32
    xp = jnp.pad(x.astype(f32), ((0, NP - N), (0, 0)))

    eye8 = jnp.eye(8, dtype=f32)
    A0s = (eye8[:, None, :] * a_src0[0][:, :, None]).reshape(128, 8)
    A0d = (eye8[:, None, :] * a_dst0[0][:, :, None]).reshape(128, 8)
    A0 = jnp.concatenate([A0s, A0d], axis=1)
    A1 = jnp.concatenate(
        [a_src1.reshape(64, 1), a_dst1.reshape(64, 1),
         jnp.zeros((64, 14), f32)], axis=1)

    ei = edge_index.astype(jnp.int32)
    loops = jnp.arange(N, dtype=jnp.int32)
    padv = jnp.full((EPAD - E_REAL,), N, jnp.int32)
    src = jnp.concatenate([ei[0], loops, padv])
    dst = jnp.concatenate([ei[1], loops, padv])
    srcb = jnp.stack([src, src + NP]).reshape(2, NBT, B)
    dstb = dst.reshape(NBT, B)

    i8 = jnp.arange(8)
    EX8 = (i8[:, None] == (jnp.arange(128)[None, :] // 16)).astype(f32)
    EX2 = (jnp.arange(4)[:, None] == 0).astype(f32) * jnp.ones((1, 64), f32)

    z1 = jnp.zeros((B, AW1), f32)
    z2 = jnp.zeros((B, AW2), f32)

    tab1, adst0, m0 = _k1(xp, W0, A0)
    mp0 = _leaky(m0[0, 0:8] + m0[0, 8:16])
    mp0r = jnp.broadcast_to(mp0.reshape(2, 4, 1), (2, 4, 16))

    tab1i = lax.bitcast_convert_type(
        tab1.astype(jnp.bfloat16).reshape(2 * NP, W1T // 2, 2), jnp.int32)
    parts1 = _edge_pass_1(tab1i, adst0, srcb, dstb, mp0r, z1)

    tab2, adst1, m1 = _k3(parts1[0], parts1[1], b0.reshape(1, 128),
                          W1, A1, EX8)
    mp1 = _leaky(m1[0, 0:1] + m1[0, 1:2])
    mp1r = jnp.broadcast_to(mp1.reshape(1, 1, 1), (2, 1, 16))

    tab2i = lax.bitcast_convert_type(
        tab2.astype(jnp.bfloat16).reshape(2 * NP, W2T // 2, 2), jnp.int32)
    parts2 = _edge_pass_2(tab2i, adst1, srcb, dstb, mp1r, z2)

    out = _k5(parts2[0], parts2[1], b1.reshape(1, 64), EX2)
    return out[:N]
